# baseline (speedup 1.0000x reference)
import functools

import jax
import jax.numpy as jnp
from jax import lax
from jax.experimental import pallas as pl
from jax.experimental.pallas import tpu as pltpu
from jax.experimental.pallas import tpu_sc as plsc

_EMB = 64
_LANE = 128


def _worker_id():
    return lax.axis_index("s") * plsc.get_sparse_core_info().num_cores + (
        lax.axis_index("c"))


@functools.lru_cache(maxsize=None)
def _build_fmt(V: int):
    info = plsc.get_sparse_core_info()
    NW = info.num_cores * info.num_subcores
    VP = ((V + _LANE - 1) // _LANE) * _LANE
    n_full = V // _LANE
    tail = V - n_full * _LANE
    per_w = -(-n_full // NW)
    per_w += per_w % 2
    n_outer = per_w // 2
    mesh = plsc.VectorSubcoreMesh(core_axis_name="c", subcore_axis_name="s")

    @functools.partial(
        pl.kernel,
        mesh=mesh,
        compiler_params=pltpu.CompilerParams(
            use_tc_tiling_on_sc=True, needs_layout_passes=False),
        out_type=jax.ShapeDtypeStruct((VP, _LANE), jnp.float32),
        scratch_types=[
            pltpu.VMEM((2, _EMB, _LANE), jnp.float32),
            pltpu.VMEM((2, _LANE, _LANE), jnp.float32),
        ]
        + [pltpu.SemaphoreType.DMA] * 4,
    )
    def k(embT_hbm, tailP_hbm, embP_hbm, tbuf, pbuf, *sems):
        gs = sems[:2]
        ws = sems[2:]
        wid = _worker_id()
        c0 = wid * per_w

        def fire_in(c, slot):
            @pl.when(c < n_full)
            def _():
                pltpu.make_async_copy(
                    embT_hbm.at[:, pl.ds(c * _LANE, _LANE)],
                    tbuf.at[slot], gs[slot]).start()

        def wait_in(c, slot):
            pltpu.make_async_copy(
                embT_hbm.at[:, pl.ds(c * _LANE, _LANE)],
                tbuf.at[slot], gs[slot]).wait()

        def fire_wb(c, slot):
            pltpu.make_async_copy(
                pbuf.at[slot], embP_hbm.at[pl.ds(c * _LANE, _LANE)],
                ws[slot]).start()

        def wait_wb(c, slot):
            @pl.when(c < n_full)
            def _():
                pltpu.make_async_copy(
                    pbuf.at[slot], embP_hbm.at[pl.ds(c * _LANE, _LANE)],
                    ws[slot]).wait()

        @pl.when(wid == 0)
        def _tail():
            pltpu.sync_copy(tailP_hbm, embP_hbm.at[pl.ds(n_full * _LANE, tail)])

        for b in range(2):
            fire_in(c0 + b, b)

        def outer(o, carry):
            for b in range(2):
                i = o * 2 + b
                c = c0 + i

                @pl.when(i >= 2)
                def _drain():
                    wait_wb(c - 2, b)

                @pl.when(c < n_full)
                def _work():
                    wait_in(c, b)

                    def t_body(t, _):
                        for e0 in range(_EMB // 16):
                            v = plsc.load_gather(
                                tbuf.at[b],
                                [jnp.arange(16, dtype=jnp.int32) + e0 * 16,
                                 jnp.full((16,), t, jnp.int32)])
                            pbuf[b, t, pl.ds(e0 * 16, 16)] = v
                        return 0

                    lax.fori_loop(0, _LANE, t_body, 0, unroll=2)

                    fire_wb(c, b)

                @pl.when(i + 2 < per_w)
                def _fire():
                    fire_in(c + 2, b)
            return carry

        lax.fori_loop(0, n_outer, outer, 0)
        for b in range(2):
            wait_wb(c0 + per_w - 2 + b, (per_w - 2 + b) % 2)

    return k


@functools.lru_cache(maxsize=None)
def _build_lookup(B: int, L: int, VP: int):
    info = plsc.get_sparse_core_info()
    NW = info.num_cores * info.num_subcores
    assert B % (NW * _LANE) == 0
    bw = B // NW
    mesh = plsc.VectorSubcoreMesh(core_axis_name="c", subcore_axis_name="s")

    @functools.partial(
        pl.kernel,
        mesh=mesh,
        compiler_params=pltpu.CompilerParams(
            use_tc_tiling_on_sc=True, needs_layout_passes=False),
        out_type=jax.ShapeDtypeStruct((L, _EMB, B), jnp.float32),
        scratch_types=[
            pltpu.VMEM((L, bw), jnp.int32),
            pltpu.VMEM((L, _LANE), jnp.float32),
            pltpu.VMEM((2, bw, _LANE), jnp.float32),
            pltpu.VMEM((2, _EMB, bw), jnp.float32),
        ]
        + [pltpu.SemaphoreType.DMA] * 4,
    )
    def k(xT_hbm, embP_hbm, posP_hbm, out_hbm, idx_v, pos_v, gbuf, slab,
          *sems):
        gs = sems[:2]
        ws = sems[2:]
        wid = _worker_id()
        b0 = wid * bw
        pltpu.sync_copy(xT_hbm.at[:, pl.ds(b0, bw)], idx_v)
        pltpu.sync_copy(posP_hbm.at[pl.ds(0, L)], pos_v)

        def gather_cp(l, slot):
            return pltpu.make_async_copy(
                embP_hbm.at[idx_v.at[l]], gbuf.at[slot], gs[slot])

        def wb_cp(l, slot):
            return pltpu.make_async_copy(
                slab.at[slot], out_hbm.at[l, :, pl.ds(b0, bw)], ws[slot])

        for b in range(2):
            gather_cp(b, b).start()

        def outer(o, carry):
            for b in range(2):
                l = o * 2 + b
                gather_cp(l, b).wait()

                @pl.when(l >= 2)
                def _drain():
                    wb_cp(l - 2, b).wait()

                def e_body(e, _):
                    pv = plsc.load_gather(
                        pos_v,
                        [jnp.full((16,), l, jnp.int32),
                         jnp.full((16,), e, jnp.int32)])
                    for j in range(bw // 16):
                        v = plsc.load_gather(
                            gbuf.at[b],
                            [jnp.arange(16, dtype=jnp.int32) + j * 16,
                             jnp.full((16,), e, jnp.int32)])
                        slab[b, e, pl.ds(j * 16, 16)] = v + pv
                    return 0

                lax.fori_loop(0, _EMB, e_body, 0, unroll=2)
                wb_cp(l, b).start()

                @pl.when(l + 2 < L)
                def _fire():
                    gather_cp(l + 2, b).start()
            return carry

        lax.fori_loop(0, L // 2, outer, 0)
        for b in range(2):
            wb_cp(L - 2 + b, b).wait()

    return k


def kernel(x, emb_table, pos_table):
    B, L = x.shape
    V, E = emb_table.shape
    assert E == _EMB
    VP = ((V + _LANE - 1) // _LANE) * _LANE
    xT = jnp.swapaxes(x, 0, 1).astype(jnp.int32)
    embT = jnp.swapaxes(emb_table, 0, 1)
    posP = jnp.pad(pos_table[:L], ((0, 0), (0, _LANE - E)))
    tailP = jnp.pad(emb_table[(V // _LANE) * _LANE:],
                    ((0, 0), (0, _LANE - E)))
    embP = _build_fmt(V)(embT, tailP)
    outT = _build_lookup(B, L, VP)(xT, embP, posP)
    return jnp.transpose(outT, (2, 0, 1))


# speedup vs baseline: 1.2159x; 1.2159x over previous
import functools

import jax
import jax.numpy as jnp
from jax import lax
from jax.experimental import pallas as pl
from jax.experimental.pallas import tpu as pltpu
from jax.experimental.pallas import tpu_sc as plsc

_EMB = 64
_LANE = 128


def _worker_id():
    return lax.axis_index("s") * plsc.get_sparse_core_info().num_cores + (
        lax.axis_index("c"))


@functools.lru_cache(maxsize=None)
def _build_fmt(V: int):
    info = plsc.get_sparse_core_info()
    NW = info.num_cores * info.num_subcores
    VP = ((V + _LANE - 1) // _LANE) * _LANE
    n_full = V // _LANE
    tail = V - n_full * _LANE
    per_w = -(-n_full // NW)
    per_w += per_w % 2
    n_outer = per_w // 2
    mesh = plsc.VectorSubcoreMesh(core_axis_name="c", subcore_axis_name="s")

    @functools.partial(
        pl.kernel,
        mesh=mesh,
        compiler_params=pltpu.CompilerParams(
            use_tc_tiling_on_sc=True, needs_layout_passes=False),
        out_type=jax.ShapeDtypeStruct((VP, _LANE), jnp.float32),
        scratch_types=[
            pltpu.VMEM((2, _EMB, _LANE), jnp.float32),
            pltpu.VMEM((2, _LANE, _LANE), jnp.float32),
        ]
        + [pltpu.SemaphoreType.DMA] * 4,
    )
    def k(embT_hbm, tailP_hbm, embP_hbm, tbuf, pbuf, *sems):
        gs = sems[:2]
        ws = sems[2:]
        wid = _worker_id()
        c0 = wid * per_w

        def fire_in(c, slot):
            @pl.when(c < n_full)
            def _():
                pltpu.make_async_copy(
                    embT_hbm.at[:, pl.ds(c * _LANE, _LANE)],
                    tbuf.at[slot], gs[slot]).start()

        def wait_in(c, slot):
            pltpu.make_async_copy(
                embT_hbm.at[:, pl.ds(c * _LANE, _LANE)],
                tbuf.at[slot], gs[slot]).wait()

        def fire_wb(c, slot):
            pltpu.make_async_copy(
                pbuf.at[slot], embP_hbm.at[pl.ds(c * _LANE, _LANE)],
                ws[slot]).start()

        def wait_wb(c, slot):
            @pl.when(c < n_full)
            def _():
                pltpu.make_async_copy(
                    pbuf.at[slot], embP_hbm.at[pl.ds(c * _LANE, _LANE)],
                    ws[slot]).wait()

        @pl.when(wid == 0)
        def _tail():
            pltpu.sync_copy(tailP_hbm, embP_hbm.at[pl.ds(n_full * _LANE, tail)])

        for b in range(2):
            fire_in(c0 + b, b)

        def outer(o, carry):
            for b in range(2):
                i = o * 2 + b
                c = c0 + i

                @pl.when(i >= 2)
                def _drain():
                    wait_wb(c - 2, b)

                @pl.when(c < n_full)
                def _work():
                    wait_in(c, b)
                    trows = [jnp.arange(16, dtype=jnp.int32) + j * 16
                             for j in range(_LANE // 16)]

                    def e_body(e, _):
                        ecol = jnp.full((16,), e, jnp.int32)
                        for j in range(_LANE // 16):
                            v = tbuf[b, e, pl.ds(j * 16, 16)]
                            plsc.store_scatter(
                                pbuf.at[b], [trows[j], ecol], v)
                        return 0

                    lax.fori_loop(0, _EMB, e_body, 0, unroll=2)

                    fire_wb(c, b)

                @pl.when(i + 2 < per_w)
                def _fire():
                    fire_in(c + 2, b)
            return carry

        lax.fori_loop(0, n_outer, outer, 0)
        for b in range(2):
            wait_wb(c0 + per_w - 2 + b, (per_w - 2 + b) % 2)

    return k


@functools.lru_cache(maxsize=None)
def _build_lookup(B: int, L: int, VP: int):
    info = plsc.get_sparse_core_info()
    NW = info.num_cores * info.num_subcores
    assert B % (NW * _LANE) == 0
    bw = B // NW
    mesh = plsc.VectorSubcoreMesh(core_axis_name="c", subcore_axis_name="s")

    @functools.partial(
        pl.kernel,
        mesh=mesh,
        compiler_params=pltpu.CompilerParams(
            use_tc_tiling_on_sc=True, needs_layout_passes=False),
        out_type=jax.ShapeDtypeStruct((L, _EMB, B), jnp.float32),
        scratch_types=[
            pltpu.VMEM((L, bw), jnp.int32),
            pltpu.VMEM((L, _LANE), jnp.float32),
            pltpu.VMEM((2, bw, _LANE), jnp.float32),
            pltpu.VMEM((2, _EMB, bw), jnp.float32),
        ]
        + [pltpu.SemaphoreType.DMA] * 4,
    )
    def k(xT_hbm, embP_hbm, posP_hbm, out_hbm, idx_v, pos_v, gbuf, slab,
          *sems):
        gs = sems[:2]
        ws = sems[2:]
        wid = _worker_id()
        b0 = wid * bw
        pltpu.sync_copy(xT_hbm.at[:, pl.ds(b0, bw)], idx_v)
        pltpu.sync_copy(posP_hbm.at[pl.ds(0, L)], pos_v)

        def gather_cp(l, slot):
            return pltpu.make_async_copy(
                embP_hbm.at[idx_v.at[l]], gbuf.at[slot], gs[slot])

        def wb_cp(l, slot):
            return pltpu.make_async_copy(
                slab.at[slot], out_hbm.at[l, :, pl.ds(b0, bw)], ws[slot])

        for b in range(2):
            gather_cp(b, b).start()

        def outer(o, carry):
            for b in range(2):
                l = o * 2 + b
                gather_cp(l, b).wait()

                @pl.when(l >= 2)
                def _drain():
                    wb_cp(l - 2, b).wait()

                erows = [jnp.arange(16, dtype=jnp.int32) + k * 16
                         for k in range(_EMB // 16)]
                pv = [pos_v[l, pl.ds(k * 16, 16)]
                      for k in range(_EMB // 16)]

                def t_body(t, _):
                    tcol = jnp.full((16,), t, jnp.int32)
                    for k in range(_EMB // 16):
                        v = gbuf[b, t, pl.ds(k * 16, 16)] + pv[k]
                        plsc.store_scatter(
                            slab.at[b], [erows[k], tcol], v)
                    return 0

                lax.fori_loop(0, bw, t_body, 0, unroll=2)
                wb_cp(l, b).start()

                @pl.when(l + 2 < L)
                def _fire():
                    gather_cp(l + 2, b).start()
            return carry

        lax.fori_loop(0, L // 2, outer, 0)
        for b in range(2):
            wb_cp(L - 2 + b, b).wait()

    return k


def kernel(x, emb_table, pos_table):
    B, L = x.shape
    V, E = emb_table.shape
    assert E == _EMB
    VP = ((V + _LANE - 1) // _LANE) * _LANE
    xT = jnp.swapaxes(x, 0, 1).astype(jnp.int32)
    embT = jnp.swapaxes(emb_table, 0, 1)
    posP = jnp.pad(pos_table[:L], ((0, 0), (0, _LANE - E)))
    tailP = jnp.pad(emb_table[(V // _LANE) * _LANE:],
                    ((0, 0), (0, _LANE - E)))
    embP = _build_fmt(V)(embT, tailP)
    outT = _build_lookup(B, L, VP)(xT, embP, posP)
    return jnp.transpose(outT, (2, 0, 1))


# speedup vs baseline: 1.9594x; 1.6115x over previous
import functools

import jax
import jax.numpy as jnp
from jax import lax
from jax.experimental import pallas as pl
from jax.experimental.pallas import tpu as pltpu
from jax.experimental.pallas import tpu_sc as plsc

_EMB = 64
_CHUNK = 128
_NB = 4
_LOOK = 2


@functools.lru_cache(maxsize=None)
def _build(BL: int, L: int):
    info = plsc.get_sparse_core_info()
    NC, NS = info.num_cores, info.num_subcores
    NW = NC * NS
    assert BL % (NW * _CHUNK) == 0
    rows_w = BL // NW
    n_chunks = rows_w // _CHUNK
    assert n_chunks % _NB == 0 and n_chunks >= 2 * _NB
    mesh = plsc.VectorSubcoreMesh(core_axis_name="c", subcore_axis_name="s")

    @functools.partial(
        pl.kernel,
        mesh=mesh,
        compiler_params=pltpu.CompilerParams(use_tc_tiling_on_sc=False),
        out_type=jax.ShapeDtypeStruct((BL, _EMB), jnp.float32),
        scratch_types=[
            pltpu.VMEM((rows_w,), jnp.int32),
            pltpu.VMEM((L, _EMB), jnp.float32),
            pltpu.VMEM((_NB, _CHUNK, _EMB), jnp.float32),
        ]
        + [pltpu.SemaphoreType.DMA] * (2 * _NB),
    )
    def k(x_hbm, emb_hbm, pos_hbm, out_hbm, idx_v, pos_v, buf_v, *sems):
        gs = sems[:_NB]
        ws = sems[_NB:]
        cid = lax.axis_index("c")
        sid = lax.axis_index("s")
        wid = sid * NC + cid
        base = wid * rows_w
        pltpu.sync_copy(pos_hbm.at[pl.ds(0, L)], pos_v)
        pltpu.sync_copy(x_hbm.at[pl.ds(base, rows_w)], idx_v)

        def gather_cp(c, slot):
            idx_view = idx_v.at[pl.ds(c * _CHUNK, _CHUNK)]
            return pltpu.make_async_copy(
                emb_hbm.at[idx_view], buf_v.at[slot], gs[slot])

        def wb_cp(c, slot):
            rb = base + c * _CHUNK
            return pltpu.make_async_copy(
                buf_v.at[slot], out_hbm.at[pl.ds(rb, _CHUNK)], ws[slot])

        for b in range(_LOOK):
            gather_cp(b, b).start()

        def outer(o, carry):
            for b in range(_NB):
                c = o * _NB + b
                gather_cp(c, b).wait()
                start = lax.rem(c * _CHUNK, L)
                bufb = buf_v.at[b]

                def row_body(j, _):
                    p = lax.rem(start + j, L)
                    for e in range(_EMB // 16):
                        sl = pl.ds(e * 16, 16)
                        plsc.addupdate(bufb.at[j, sl], pos_v[p, sl])
                    return 0

                lax.fori_loop(0, _CHUNK, row_body, 0, unroll=2)
                wb_cp(c, b).start()
                b2 = (b + _LOOK) % _NB

                @pl.when(c >= _NB - _LOOK)
                def _drain():
                    wb_cp(c - (_NB - _LOOK), b2).wait()

                @pl.when(c + _LOOK < n_chunks)
                def _fire():
                    gather_cp(c + _LOOK, b2).start()
            return carry

        lax.fori_loop(0, n_chunks // _NB, outer, 0)
        for c in range(n_chunks - (_NB - _LOOK), n_chunks):
            wb_cp(c, c % _NB).wait()

    return k


def kernel(x, emb_table, pos_table):
    B, L = x.shape
    BL = B * L
    xf = x.reshape(BL).astype(jnp.int32)
    out = _build(BL, L)(xf, emb_table, pos_table)
    return out.reshape(B, L, _EMB)
